# Initial kernel scaffold; baseline (speedup 1.0000x reference)
import jax
import jax.numpy as jnp
from jax.experimental import pallas as pl


def kernel(class_outs, regression_outs, boxes, images_hw):
    raise NotImplementedError("write your pallas kernel here")


# speedup vs baseline: 2.2651x; 2.2651x over previous
import math

import jax
import jax.numpy as jnp
from jax.experimental import pallas as pl

_N = 5000
_C = 80
_SCORE_THR = 0.05
_IOU_THR = 0.5
_MAX_PER_IMG = 100
_PRE_NMS = 1000
_PAD = 1024
_MAX_RATIO = float(abs(math.log(16.0 / 1000.0)))


def _softmax_mask_kernel(x_ref, o_ref):
    x = x_ref[:]
    m = jnp.max(x, axis=1, keepdims=True)
    e = jnp.exp(x - m)
    s = jnp.sum(e, axis=1, keepdims=True)
    fg = e[:, :_C] / s
    o_ref[:] = jnp.where(fg > _SCORE_THR, fg, -1.0)


def _decode_nms_kernel(d0, d1, d2, d3, p0, p1, p2, p3, sc, lb, hb, wb,
                       x1o, y1o, x2o, y2o, fo):
    dx = d0[:] * 0.1
    dy = d1[:] * 0.1
    dw = jnp.clip(d2[:] * 0.2, -_MAX_RATIO, _MAX_RATIO)
    dh = jnp.clip(d3[:] * 0.2, -_MAX_RATIO, _MAX_RATIO)
    pw = p2[:] - p0[:]
    ph = p3[:] - p1[:]
    pcx = (p0[:] + p2[:]) * 0.5
    pcy = (p1[:] + p3[:]) * 0.5
    gcx = pcx + pw * dx
    gcy = pcy + ph * dy
    gw = pw * jnp.exp(dw)
    gh = ph * jnp.exp(dh)
    H = hb[:]
    W = wb[:]
    x1 = jnp.clip(gcx - gw * 0.5, 0.0, W)
    y1 = jnp.clip(gcy - gh * 0.5, 0.0, H)
    x2 = jnp.clip(gcx + gw * 0.5, 0.0, W)
    y2 = jnp.clip(gcy + gh * 0.5, 0.0, H)
    x1o[:] = x1
    y1o[:] = y1
    x2o[:] = x2
    y2o[:] = y2

    off = lb[:] * (jnp.maximum(H, W) + 1.0)
    bx1 = x1 + off
    by1 = y1 + off
    bx2 = x2 + off
    by2 = y2 + off
    area = (bx2 - bx1) * (by2 - by1)
    scores = sc[:]
    keep0 = jnp.where(scores > _SCORE_THR, 1.0, 0.0)

    row = jax.lax.broadcasted_iota(jnp.int32, (8, 128), 0)
    col = jax.lax.broadcasted_iota(jnp.int32, (8, 128), 1)
    idx = row * 128 + col

    def body(i, keep):
        ohf = jnp.where(idx == i, 1.0, 0.0)
        ki = jnp.sum(ohf * keep)
        xi1 = jnp.sum(ohf * bx1)
        yi1 = jnp.sum(ohf * by1)
        xi2 = jnp.sum(ohf * bx2)
        yi2 = jnp.sum(ohf * by2)
        ai = jnp.sum(ohf * area)
        iw = jnp.maximum(jnp.minimum(bx2, xi2) - jnp.maximum(bx1, xi1), 0.0)
        ih = jnp.maximum(jnp.minimum(by2, yi2) - jnp.maximum(by1, yi1), 0.0)
        inter = iw * ih
        iou = inter / jnp.maximum(area + ai - inter, 1e-6)
        sup = jnp.where((iou > _IOU_THR) & (idx > i), 1.0, 0.0)
        sup = sup * jnp.where(ki > 0.0, 1.0, 0.0)
        return keep * (1.0 - sup)

    keep = jax.lax.fori_loop(0, _PRE_NMS, body, keep0)
    fo[:] = jnp.where(keep > 0.0, scores, -1.0)


def kernel(class_outs, regression_outs, boxes, images_hw):
    f32 = jnp.float32
    hw = images_hw.astype(f32)
    H = hw[0, 0]
    W = hw[0, 1]

    masked = pl.pallas_call(
        _softmax_mask_kernel,
        out_shape=jax.ShapeDtypeStruct((_N, _C), f32),
    )(class_outs)

    flat = masked.reshape(-1)
    top_scores, idx = jax.lax.top_k(flat, _PRE_NMS)
    rows = idx // _C
    cls = (idx - rows * _C).astype(jnp.int32)
    deltas = regression_outs.reshape(-1, 4)[idx]
    props = boxes[rows, :4]

    pad = _PAD - _PRE_NMS
    d = jnp.pad(deltas, ((0, pad), (0, 0)))
    p = jnp.pad(props, ((0, pad), (0, 0)))
    sc = jnp.pad(top_scores, (0, pad), constant_values=-1.0)
    lb = jnp.pad(cls.astype(f32), (0, pad))

    shp = (8, 128)
    args = [d[:, k].reshape(shp) for k in range(4)]
    args += [p[:, k].reshape(shp) for k in range(4)]
    args += [sc.reshape(shp), lb.reshape(shp),
             jnp.full(shp, H, f32), jnp.full(shp, W, f32)]

    outs = pl.pallas_call(
        _decode_nms_kernel,
        out_shape=[jax.ShapeDtypeStruct(shp, f32)] * 5,
    )(*args)

    x1, y1, x2, y2, final = [o.reshape(-1)[:_PRE_NMS] for o in outs]
    cand_boxes = jnp.stack([x1, y1, x2, y2], axis=1)
    det_scores, didx = jax.lax.top_k(final, _MAX_PER_IMG)
    det_boxes = cand_boxes[didx]
    det_labels = cls[didx]
    return det_boxes, det_scores, det_labels
